# speedup vs baseline: 3.0168x; 3.0168x over previous
import jax
import jax.numpy as jnp
from jax.experimental import pallas as pl
from jax.experimental.pallas import tpu as pltpu

E = 64
K = 2
D = 1024
F = 768
T = 2048
A = T * K
R = 64
NPAD = A + E * (R - 1)
NT = NPAD // R
TB = 256


def _router_body(x_ref, gw_ref, idx_ref, w_ref):
    logits = jnp.dot(x_ref[...], gw_ref[...],
                     preferred_element_type=jnp.float32)
    iot = jax.lax.broadcasted_iota(jnp.int32, logits.shape, 1)
    m1 = jnp.max(logits, axis=1, keepdims=True)
    a1 = jnp.min(jnp.where(logits == m1, iot, E), axis=1, keepdims=True)
    l2 = jnp.where(iot == a1, -jnp.inf, logits)
    m2 = jnp.max(l2, axis=1, keepdims=True)
    a2 = jnp.min(jnp.where(l2 == m2, iot, E), axis=1, keepdims=True)
    w1 = 1.0 / (1.0 + jnp.exp(m2 - m1))
    idx_ref[...] = jnp.concatenate([a1, a2], axis=1).astype(jnp.int32)
    w_ref[...] = jnp.concatenate([w1, 1.0 - w1], axis=1)


def _router(x, gate_w):
    return pl.pallas_call(
        _router_body,
        grid=(T // TB,),
        in_specs=[
            pl.BlockSpec((TB, D), lambda t: (t, 0)),
            pl.BlockSpec((D, E), lambda t: (0, 0)),
        ],
        out_specs=[
            pl.BlockSpec((TB, K), lambda t: (t, 0)),
            pl.BlockSpec((TB, K), lambda t: (t, 0)),
        ],
        out_shape=[
            jax.ShapeDtypeStruct((T, K), jnp.int32),
            jax.ShapeDtypeStruct((T, K), jnp.float32),
        ],
    )(x, gate_w)


def _dispatch_metadata(topk_idx, topk_w):
    flat_e = topk_idx.reshape(-1)
    flat_w = topk_w.reshape(-1)
    flat_t = jnp.arange(A, dtype=jnp.int32) // K
    order = jnp.argsort(flat_e)
    sorted_e = flat_e[order]
    counts = jnp.zeros((E,), jnp.int32).at[flat_e].add(1)
    padded = ((counts + R - 1) // R) * R
    pad_end = jnp.cumsum(padded)
    pad_start = pad_end - padded
    start = jnp.cumsum(counts) - counts
    pos_sorted = pad_start[sorted_e] + jnp.arange(A, dtype=jnp.int32) - start[sorted_e]
    row_token = jnp.zeros((NPAD,), jnp.int32).at[pos_sorted].set(flat_t[order])
    row_weight = jnp.zeros((NPAD, 1), jnp.float32).at[pos_sorted, 0].set(flat_w[order])
    tile_expert = jnp.minimum(
        jnp.searchsorted(pad_end, jnp.arange(NT, dtype=jnp.int32) * R, side="right"),
        E - 1,
    ).astype(jnp.int32)
    pos = jnp.zeros((A,), jnp.int32).at[order].set(pos_sorted).reshape(T, K)
    return tile_expert, row_token, row_weight, pos[:, 0], pos[:, 1]


def _moe_body(te_ref, xs_ref, wgu_ref, wd_ref, rw_ref, ys_ref):
    xb = xs_ref[...].astype(jnp.bfloat16)
    wgu = wgu_ref[0].astype(jnp.bfloat16)
    gu = jnp.dot(xb, wgu, preferred_element_type=jnp.float32)
    g = gu[:, :F]
    u = gu[:, F:]
    h = (g * jax.lax.logistic(g)) * u
    yd = jnp.dot(h.astype(jnp.bfloat16), wd_ref[0].astype(jnp.bfloat16),
                 preferred_element_type=jnp.float32)
    ys_ref[...] = yd * rw_ref[...]


def _grouped_mlp(tile_expert, xs, w_gate_up, w_down, row_weight):
    grid_spec = pltpu.PrefetchScalarGridSpec(
        num_scalar_prefetch=1,
        grid=(NT,),
        in_specs=[
            pl.BlockSpec((R, D), lambda t, te: (t, 0)),
            pl.BlockSpec((1, D, 2 * F), lambda t, te: (te[t], 0, 0)),
            pl.BlockSpec((1, F, D), lambda t, te: (te[t], 0, 0)),
            pl.BlockSpec((R, 1), lambda t, te: (t, 0)),
        ],
        out_specs=pl.BlockSpec((R, D), lambda t, te: (t, 0)),
    )
    return pl.pallas_call(
        _moe_body,
        grid_spec=grid_spec,
        out_shape=jax.ShapeDtypeStruct((NPAD, D), jnp.float32),
    )(tile_expert, xs, w_gate_up, w_down, row_weight)


def kernel(hidden_states, gate_w, w_gate_up, w_down):
    x = hidden_states
    topk_idx, topk_w = _router(x, gate_w)
    tile_expert, row_token, row_weight, pos1, pos2 = _dispatch_metadata(
        topk_idx, topk_w)
    xs = jnp.take(x, row_token, axis=0)
    ys = _grouped_mlp(tile_expert, xs, w_gate_up, w_down, row_weight)
    return ys[pos1] + ys[pos2]


# speedup vs baseline: 3.1150x; 1.0325x over previous
import jax
import jax.numpy as jnp
from jax.experimental import pallas as pl
from jax.experimental.pallas import tpu as pltpu
from jax.experimental.pallas import tpu_sc as plsc

E = 64
K = 2
D = 1024
F = 768
T = 2048
A = T * K
R = 64
NPAD = A + E * (R - 1)
NT = NPAD // R
TB = 256
CW = 16


def _router_body(x_ref, gw_ref, idx_ref, w_ref):
    logits = jnp.dot(x_ref[...], gw_ref[...],
                     preferred_element_type=jnp.float32)
    iot = jax.lax.broadcasted_iota(jnp.int32, logits.shape, 1)
    m1 = jnp.max(logits, axis=1, keepdims=True)
    a1 = jnp.min(jnp.where(logits == m1, iot, E), axis=1, keepdims=True)
    l2 = jnp.where(iot == a1, -jnp.inf, logits)
    m2 = jnp.max(l2, axis=1, keepdims=True)
    a2 = jnp.min(jnp.where(l2 == m2, iot, E), axis=1, keepdims=True)
    w1 = 1.0 / (1.0 + jnp.exp(m2 - m1))
    idx_ref[...] = jnp.concatenate([a1, a2], axis=1).astype(jnp.int32)
    w_ref[...] = jnp.concatenate([w1, 1.0 - w1], axis=1)


def _router(x, gate_w):
    return pl.pallas_call(
        _router_body,
        grid=(T // TB,),
        in_specs=[
            pl.BlockSpec((TB, D), lambda t: (t, 0)),
            pl.BlockSpec((D, E), lambda t: (0, 0)),
        ],
        out_specs=[
            pl.BlockSpec((TB, K), lambda t: (t, 0)),
            pl.BlockSpec((TB, K), lambda t: (t, 0)),
        ],
        out_shape=[
            jax.ShapeDtypeStruct((T, K), jnp.int32),
            jax.ShapeDtypeStruct((T, K), jnp.float32),
        ],
    )(x, gate_w)


def _dispatch_metadata(topk_idx, topk_w):
    flat_e = topk_idx.reshape(-1)
    flat_w = topk_w.reshape(-1)
    flat_t = jnp.arange(A, dtype=jnp.int32) // K
    order = jnp.argsort(flat_e)
    sorted_e = flat_e[order]
    counts = jnp.zeros((E,), jnp.int32).at[flat_e].add(1)
    padded = ((counts + R - 1) // R) * R
    pad_end = jnp.cumsum(padded)
    pad_start = pad_end - padded
    start = jnp.cumsum(counts) - counts
    pos_sorted = pad_start[sorted_e] + jnp.arange(A, dtype=jnp.int32) - start[sorted_e]
    row_token = jnp.zeros((NPAD, 1), jnp.int32).at[pos_sorted, 0].set(flat_t[order])
    row_weight = jnp.zeros((NPAD, 1), jnp.float32).at[pos_sorted, 0].set(flat_w[order])
    tile_expert = jnp.minimum(
        jnp.searchsorted(pad_end, jnp.arange(NT, dtype=jnp.int32) * R, side="right"),
        E - 1,
    ).astype(jnp.int32)
    pos = jnp.zeros((A,), jnp.int32).at[order].set(pos_sorted).reshape(T, K)
    return tile_expert, row_token, row_weight, pos[:, 0], pos[:, 1]


def _moe_body(te_ref, x_ref, rt_ref, wgu_ref, wd_ref, rw_ref, ys_ref):
    rt = rt_ref[...]
    iota_t = jax.lax.broadcasted_iota(jnp.int32, (R, T), 1)
    onehot = (rt == iota_t).astype(jnp.bfloat16)
    xb = jnp.dot(onehot, x_ref[...],
                 preferred_element_type=jnp.float32)
    wgu = wgu_ref[0].astype(jnp.bfloat16)
    gu = jnp.dot(xb.astype(jnp.bfloat16), wgu,
                 preferred_element_type=jnp.float32)
    g = gu[:, :F]
    u = gu[:, F:]
    h = (g * jax.lax.logistic(g)) * u
    yd = jnp.dot(h.astype(jnp.bfloat16), wd_ref[0].astype(jnp.bfloat16),
                 preferred_element_type=jnp.float32)
    ys_ref[...] = yd * rw_ref[...]


def _grouped_mlp(tile_expert, x_bf16, row_token, w_gate_up, w_down, row_weight):
    grid_spec = pltpu.PrefetchScalarGridSpec(
        num_scalar_prefetch=1,
        grid=(NT,),
        in_specs=[
            pl.BlockSpec((T, D), lambda t, te: (0, 0)),
            pl.BlockSpec((R, 1), lambda t, te: (t, 0)),
            pl.BlockSpec((1, D, 2 * F), lambda t, te: (te[t], 0, 0)),
            pl.BlockSpec((1, F, D), lambda t, te: (te[t], 0, 0)),
            pl.BlockSpec((R, 1), lambda t, te: (t, 0)),
        ],
        out_specs=pl.BlockSpec((R, D), lambda t, te: (t, 0)),
    )
    return pl.pallas_call(
        _moe_body,
        grid_spec=grid_spec,
        out_shape=jax.ShapeDtypeStruct((NPAD, D), jnp.float32),
        compiler_params=pltpu.CompilerParams(
            dimension_semantics=("parallel",)),
    )(tile_expert, x_bf16, row_token, w_gate_up, w_down, row_weight)


def _sc_combine(ys, pos1, pos2):
    p1 = pos1.reshape(1, T)
    p2 = pos2.reshape(1, T)
    mesh = plsc.VectorSubcoreMesh(core_axis_name="c", subcore_axis_name="s")

    @pl.kernel(
        out_type=jax.ShapeDtypeStruct((T, D), jnp.float32),
        mesh=mesh,
        scratch_types=[pltpu.VMEM((CW, D), jnp.float32),
                       pltpu.VMEM((CW, D), jnp.float32)],
    )
    def combine_kernel(ys_hbm, p1_hbm, p2_hbm, o_hbm, g1, g2):
        def body(p1_vmem, p2_vmem, o_vmem):
            pltpu.sync_copy(ys_hbm.at[p1_vmem.at[0]], g1)
            pltpu.sync_copy(ys_hbm.at[p2_vmem.at[0]], g2)

            @pl.loop(0, CW)
            def _(r):
                @pl.loop(0, D, step=16)
                def _(c):
                    slc = (pl.ds(r, 1), pl.ds(c, 16))
                    o_vmem.at[slc][...] = g1.at[slc][...] + g2.at[slc][...]

        pltpu.emit_pipeline(
            body,
            grid=(T // CW,),
            in_specs=[pl.BlockSpec((1, CW), index_map=lambda i: (0, i)),
                      pl.BlockSpec((1, CW), index_map=lambda i: (0, i))],
            out_specs=[pl.BlockSpec((CW, D), index_map=lambda i: (i, 0))],
            core_axis_name=("c", "s"),
            dimension_semantics=(pltpu.PARALLEL,),
        )(p1_hbm, p2_hbm, o_hbm)

    return combine_kernel(ys, p1, p2)


def kernel(hidden_states, gate_w, w_gate_up, w_down):
    x = hidden_states
    topk_idx, topk_w = _router(x, gate_w)
    tile_expert, row_token, row_weight, pos1, pos2 = _dispatch_metadata(
        topk_idx, topk_w)
    ys = _grouped_mlp(tile_expert, x.astype(jnp.bfloat16), row_token,
                      w_gate_up, w_down, row_weight)
    return ys[pos1] + ys[pos2]
